# baseline (speedup 1.0000x reference)
import functools

import jax
import jax.numpy as jnp
from jax import lax
from jax.experimental import pallas as pl
from jax.experimental.pallas import tpu as pltpu
from jax.experimental.pallas import tpu_sc as plsc

NW = 32
CHUNK = 1024
IDXW = 128
SUB = CHUNK // IDXW
NBUF = 3


def _sc_gather(idx2d, table, n, d):
    per_w = n // NW
    n_it = per_w // CHUNK
    idx_rows = per_w // IDXW
    mesh = plsc.VectorSubcoreMesh(core_axis_name="c", subcore_axis_name="s")

    @functools.partial(
        pl.kernel,
        mesh=mesh,
        out_type=jax.ShapeDtypeStruct((n, d), jnp.float32),
        scratch_types=(
            [pltpu.VMEM((idx_rows, IDXW), jnp.int32)]
            + [pltpu.VMEM((CHUNK, d), jnp.float32) for _ in range(NBUF)]
            + [pltpu.SemaphoreType.DMA for _ in range(2 * NBUF + 1)]
        ),
        compiler_params=pltpu.CompilerParams(use_tc_tiling_on_sc=False),
    )
    def k(idx_hbm, table_hbm, out_hbm, idx_v, *scratch):
        rows = scratch[:NBUF]
        si = scratch[NBUF]
        sgs = scratch[NBUF + 1 : NBUF + 1 + NBUF]
        sos = scratch[NBUF + 1 + NBUF :]
        wid = lax.axis_index("s") * 2 + lax.axis_index("c")
        base = wid * per_w
        bufs = tuple(zip(rows, sgs, sos))

        def out_slice(i):
            off = pl.multiple_of(base + i * CHUNK, CHUNK)
            return out_hbm.at[pl.ds(off, CHUNK)]

        irow = pl.multiple_of(base // IDXW, SUB)
        idx_src = idx_hbm.at[pl.ds(irow, idx_rows)]
        pltpu.async_copy(idx_src, idx_v, si)
        pltpu.make_async_copy(idx_src, idx_v, si).wait()

        def gather_cp(i, j, rows_v, sg):
            return pltpu.make_async_copy(
                table_hbm.at[idx_v.at[i * SUB + j]],
                rows_v.at[pl.ds(j * IDXW, IDXW)],
                sg,
            )

        def start(i):
            rows_v, sg, so = bufs[i % NBUF]
            if i >= NBUF:
                pltpu.make_async_copy(rows_v, out_slice(i - NBUF), so).wait()
            for j in range(SUB):
                gather_cp(i, j, rows_v, sg).start()

        def finish(i):
            rows_v, sg, so = bufs[i % NBUF]
            for j in range(SUB):
                gather_cp(i, j, rows_v, sg).wait()
            pltpu.async_copy(rows_v, out_slice(i), so)

        for i in range(min(NBUF - 1, n_it)):
            start(i)
        for i in range(n_it):
            if i + NBUF - 1 < n_it:
                start(i + NBUF - 1)
            finish(i)

        for i in range(max(0, n_it - NBUF), n_it):
            rows_v, _, so = bufs[i % NBUF]
            pltpu.make_async_copy(rows_v, out_slice(i), so).wait()

    return k(idx2d, table)


def kernel(words_seq, table):
    b, l = words_seq.shape
    v, d = table.shape
    n = b * l
    idx2d = words_seq.astype(jnp.int32).reshape(n // IDXW, IDXW)
    out = _sc_gather(idx2d, table, n, d)
    return out.reshape(b, l, d)
